# Initial kernel scaffold; baseline (speedup 1.0000x reference)
import jax
import jax.numpy as jnp
from jax.experimental import pallas as pl


def kernel(h, edge_index, edge_weight, lamda, W):
    raise NotImplementedError("write your pallas kernel here")


# speedup vs baseline: 3.7324x; 3.7324x over previous
import functools

import jax
import jax.numpy as jnp
from jax import lax
from jax.experimental import pallas as pl
from jax.experimental.pallas import tpu as pltpu
from jax.experimental.pallas import tpu_sc as plsc

N = 10000
E = 320000
D = 128
K = 10
ALPHA = 0.1

NC = 2
NS = 16
NW = NC * NS
CB = 128
NCH = 79
PER = NCH * CB
ROWS = 10240
RPT = ROWS // NS
LANES = 16
BLK = 400
GRID = N // BLK


def _transform_body(lam_ref, h_ref, w_ref, out_ref):
    x = jnp.dot(h_ref[...], w_ref[...], preferred_element_type=jnp.float32)
    elu = jnp.where(x > 0.0, x, jnp.exp(x) - 1.0)
    lam = lam_ref[0]
    out_ref[...] = lam * elu + (1.0 - lam) * h_ref[...]


def _merge_body(p_ref, h_ref, out_ref):
    out_ref[...] = (1.0 - ALPHA) * (p_ref[0] + p_ref[1]) + ALPHA * h_ref[...]


_mesh = plsc.VectorSubcoreMesh(core_axis_name="c", subcore_axis_name="s")


@functools.partial(
    pl.kernel,
    mesh=_mesh,
    out_type=jax.ShapeDtypeStruct((NC, ROWS, D), jnp.float32),
    scratch_types=[
        pltpu.VMEM((NCH, CB), jnp.int32),
        pltpu.VMEM((NCH, CB), jnp.int32),
        pltpu.VMEM((PER,), jnp.float32),
        pltpu.VMEM((CB, D), jnp.float32),
        pltpu.VMEM_SHARED((ROWS, D), jnp.float32),
        pltpu.SemaphoreType.DMA,
    ],
)
def _spmm(hp_hbm, src_hbm, dst_hbm, ew_hbm, out_hbm,
          src_v, dst_v, ew_v, gbuf, acc, sem):
    c = lax.axis_index("c")
    s = lax.axis_index("s")
    wid = s * NC + c

    pltpu.sync_copy(src_hbm.at[wid], src_v)
    pltpu.sync_copy(dst_hbm.at[wid], dst_v)
    pltpu.sync_copy(ew_hbm.at[wid], ew_v)

    zero = jnp.zeros((LANES,), jnp.float32)

    def _zrow(i, carry):
        for j in range(D // LANES):
            gbuf[i, pl.ds(j * LANES, LANES)] = zero
        return carry

    lax.fori_loop(0, CB, _zrow, 0)
    for b in range(RPT // CB):
        pltpu.sync_copy(gbuf, acc.at[pl.ds(s * RPT + b * CB, CB)])
    plsc.subcore_barrier()

    def _chunk(ch, carry):
        pltpu.async_copy(hp_hbm.at[src_v.at[ch]], gbuf, sem).wait()

        def _group(g, c2):
            wgrp = ew_v[pl.ds(ch * CB + g * LANES, LANES)]
            for e in range(LANES):
                wv = wgrp.at[jnp.full((LANES,), e, jnp.int32)].get(
                    mode="promise_in_bounds")
                row = g * LANES + e
                for j in range(D // LANES):
                    gbuf[row, pl.ds(j * LANES, LANES)] = (
                        gbuf[row, pl.ds(j * LANES, LANES)] * wv)
            return c2

        lax.fori_loop(0, CB // LANES, _group, 0)
        pltpu.sync_copy(gbuf, acc.at[dst_v.at[ch]], add=True)
        return carry

    lax.fori_loop(0, NCH, _chunk, 0)
    plsc.subcore_barrier()

    pltpu.sync_copy(acc.at[pl.ds(s * RPT, RPT)],
                    out_hbm.at[c, pl.ds(s * RPT, RPT)])


def _transform(lamda, h, W):
    return pl.pallas_call(
        _transform_body,
        grid=(GRID,),
        in_specs=[
            pl.BlockSpec(memory_space=pltpu.SMEM),
            pl.BlockSpec((BLK, D), lambda i: (i, 0)),
            pl.BlockSpec((D, D), lambda i: (0, 0)),
        ],
        out_specs=pl.BlockSpec((BLK, D), lambda i: (i, 0)),
        out_shape=jax.ShapeDtypeStruct((N, D), jnp.float32),
    )(lamda, h, W)


def _merge(p, h):
    return pl.pallas_call(
        _merge_body,
        grid=(GRID,),
        in_specs=[
            pl.BlockSpec((NC, BLK, D), lambda i: (0, i, 0)),
            pl.BlockSpec((BLK, D), lambda i: (i, 0)),
        ],
        out_specs=pl.BlockSpec((BLK, D), lambda i: (i, 0)),
        out_shape=jax.ShapeDtypeStruct((N, D), jnp.float32),
    )(p, h)


def kernel(h, edge_index, edge_weight, lamda, W):
    dst = edge_index[0].astype(jnp.int32)
    src = edge_index[1].astype(jnp.int32)
    pad = NW * PER - E
    src_p = jnp.concatenate([src, jnp.zeros((pad,), jnp.int32)])
    dst_p = jnp.concatenate([dst, jnp.zeros((pad,), jnp.int32)])
    ew_p = jnp.concatenate([edge_weight.astype(jnp.float32),
                            jnp.zeros((pad,), jnp.float32)])
    src_p = src_p.reshape(NW, NCH, CB)
    dst_p = dst_p.reshape(NW, NCH, CB)
    ew_p = ew_p.reshape(NW, PER)

    hp = _transform(lamda, h, W)
    for _ in range(K):
        p = _spmm(hp, src_p, dst_p, ew_p)
        hp = _merge(p, h)
    return hp
